# baseline (speedup 1.0000x reference)
import jax
import jax.numpy as jnp
from jax.experimental import pallas as pl

_N = 4096
_NFEAT = 128
_NHID = 64
_H = 4
_P = 1024
_ALPHA = 0.2
_BR = 256
_BP = 256
_LOG2E = 1.4426950408889634


def _elu(v):
    return jnp.where(v > 0.0, v, jnp.exp(v) - 1.0)


def _aug(wh):
    r = wh.shape[0]
    return jnp.concatenate(
        [wh, jnp.ones((r, 1), jnp.float32), jnp.zeros((r, _NHID - 1), jnp.float32)],
        axis=1)


def _proj_body(x_ref, ws_ref, wh_ref):
    xb = x_ref[...]
    for h in range(_H):
        wh_ref[h] = _aug(jnp.dot(xb, ws_ref[h], preferred_element_type=jnp.float32))


def _masked_softmax_matmul(maskf, f1, f2, wh_aug):
    m2 = jnp.max(f2, axis=1, keepdims=True)
    z0 = f1 + m2
    mrow = jnp.maximum(z0, _ALPHA * z0)
    ra = jnp.exp(z0 - mrow)
    rc = jnp.exp(_ALPHA * z0 - mrow)
    cb = jnp.exp(f2 - m2)
    cd = jnp.exp(_ALPHA * (f2 - m2))
    q = jnp.maximum(ra * cb, rc * cd)
    p = q * maskf
    out = jnp.dot(p, wh_aug, preferred_element_type=jnp.float32)
    return out[:, :_NHID] / out[:, _NHID:_NHID + 1]


def _layer1_body(adj_ref, wh_ref, as_ref, wout_ref, who_ref):
    i = pl.program_id(0)
    maskf = jnp.where(adj_ref[...] > 0.0, 1.0, 0.0)
    parts = []
    for h in range(_H):
        wh_aug = wh_ref[h]
        wh_blk = wh_ref[h, pl.ds(i * _BR, _BR), :_NHID]
        a1 = as_ref[h, :_NHID, :]
        a2 = as_ref[h, _NHID:, :]
        f1 = jnp.dot(wh_blk, a1, preferred_element_type=jnp.float32)
        f2 = jax.lax.dot_general(a2, wh_ref[h, :, :_NHID],
                                 (((0,), (1,)), ((), ())),
                                 preferred_element_type=jnp.float32)
        hp = _masked_softmax_matmul(maskf, f1, f2, wh_aug)
        parts.append(_elu(hp))
    hcat = jnp.concatenate(parts, axis=1)
    who_ref[...] = _aug(jnp.dot(hcat, wout_ref[...], preferred_element_type=jnp.float32))


def _layer2_body(adj_ref, who_ref, aout_ref, h2_ref):
    i = pl.program_id(0)
    maskf = jnp.where(adj_ref[...] > 0.0, 1.0, 0.0)
    who_aug = who_ref[...]
    who_blk = who_ref[pl.ds(i * _BR, _BR), :_NHID]
    a1 = aout_ref[:_NHID, :]
    a2 = aout_ref[_NHID:, :]
    f1 = jnp.dot(who_blk, a1, preferred_element_type=jnp.float32)
    f2 = jax.lax.dot_general(a2, who_ref[:, :_NHID],
                             (((0,), (1,)), ((), ())),
                             preferred_element_type=jnp.float32)
    hp = _masked_softmax_matmul(maskf, f1, f2, who_aug)
    h2_ref[...] = _elu(hp)


def _score_body(p1_ref, p2_ref, h2_ref, w_ref, out_ref):
    h2 = h2_ref[...]
    e1 = jnp.dot(p1_ref[...], h2, preferred_element_type=jnp.float32)
    e2 = jnp.dot(p2_ref[...], h2, preferred_element_type=jnp.float32)
    t = jnp.dot(e1, w_ref[...], preferred_element_type=jnp.float32)
    out_ref[...] = jnp.sum(t * e2, axis=1, keepdims=True)


def kernel(x, adj, pair1_map, pair2_map, Ws, As, W_out, A_out, weight):
    wh = pl.pallas_call(
        _proj_body,
        grid=(_N // _BR,),
        in_specs=[
            pl.BlockSpec((_BR, _NFEAT), lambda i: (i, 0)),
            pl.BlockSpec((_H, _NFEAT, _NHID), lambda i: (0, 0, 0)),
        ],
        out_specs=pl.BlockSpec((_H, _BR, 2 * _NHID), lambda i: (0, i, 0)),
        out_shape=jax.ShapeDtypeStruct((_H, _N, 2 * _NHID), jnp.float32),
    )(x, Ws)

    who = pl.pallas_call(
        _layer1_body,
        grid=(_N // _BR,),
        in_specs=[
            pl.BlockSpec((_BR, _N), lambda i: (i, 0)),
            pl.BlockSpec((_H, _N, 2 * _NHID), lambda i: (0, 0, 0)),
            pl.BlockSpec((_H, 2 * _NHID, 1), lambda i: (0, 0, 0)),
            pl.BlockSpec((_H * _NHID, _NHID), lambda i: (0, 0)),
        ],
        out_specs=pl.BlockSpec((_BR, 2 * _NHID), lambda i: (i, 0)),
        out_shape=jax.ShapeDtypeStruct((_N, 2 * _NHID), jnp.float32),
    )(adj, wh, As, W_out)

    h2 = pl.pallas_call(
        _layer2_body,
        grid=(_N // _BR,),
        in_specs=[
            pl.BlockSpec((_BR, _N), lambda i: (i, 0)),
            pl.BlockSpec((_N, 2 * _NHID), lambda i: (0, 0)),
            pl.BlockSpec((2 * _NHID, 1), lambda i: (0, 0)),
        ],
        out_specs=pl.BlockSpec((_BR, _NHID), lambda i: (i, 0)),
        out_shape=jax.ShapeDtypeStruct((_N, _NHID), jnp.float32),
    )(adj, who, A_out)

    scores = pl.pallas_call(
        _score_body,
        grid=(_P // _BP,),
        in_specs=[
            pl.BlockSpec((_BP, _N), lambda i: (i, 0)),
            pl.BlockSpec((_BP, _N), lambda i: (i, 0)),
            pl.BlockSpec((_N, _NHID), lambda i: (0, 0)),
            pl.BlockSpec((_NHID, _NHID), lambda i: (0, 0)),
        ],
        out_specs=pl.BlockSpec((_BP, 1), lambda i: (i, 0)),
        out_shape=jax.ShapeDtypeStruct((_P, 1), jnp.float32),
    )(pair1_map, pair2_map, h2, weight)

    return scores.reshape(_P)


# speedup vs baseline: 1.2879x; 1.2879x over previous
import jax
import jax.numpy as jnp
from jax.experimental import pallas as pl

_N = 4096
_NFEAT = 128
_NHID = 64
_H = 4
_P = 1024
_ALPHA = 0.2
_BR = 512
_BP = 512


def _elu(v):
    return jnp.where(v > 0.0, v, jnp.exp(v) - 1.0)


def _aug(wh):
    r = wh.shape[0]
    return jnp.concatenate(
        [wh, jnp.ones((r, 1), jnp.float32), jnp.zeros((r, _NHID - 1), jnp.float32)],
        axis=1)


def _proj_body(x_ref, ws_ref, wh_ref, whb_ref):
    xb = x_ref[...]
    for h in range(_H):
        wh = jnp.dot(xb, ws_ref[h], preferred_element_type=jnp.float32)
        wh_ref[h] = wh
        whb_ref[h] = _aug(wh).astype(jnp.bfloat16)


def _masked_softmax_matmul(maskb, f1, f2, wh_aug):
    m2 = jnp.max(f2, axis=1, keepdims=True)
    z0 = f1 + m2
    mrow = jnp.maximum(z0, _ALPHA * z0)
    ra = jnp.exp(z0 - mrow).astype(jnp.bfloat16)
    rc = jnp.exp(_ALPHA * z0 - mrow).astype(jnp.bfloat16)
    cb = jnp.exp(f2 - m2).astype(jnp.bfloat16)
    cd = jnp.exp(_ALPHA * (f2 - m2)).astype(jnp.bfloat16)
    q = jnp.maximum(ra * cb, rc * cd)
    p = q * maskb
    out = jnp.dot(p, wh_aug, preferred_element_type=jnp.float32)
    return out[:, :_NHID] / out[:, _NHID:_NHID + 1]


def _layer1_body(adj_ref, wh_ref, whb_ref, as_ref, wout_ref,
                 who_ref, whob_ref, mask8_ref):
    i = pl.program_id(0)
    ab = adj_ref[...].astype(jnp.bfloat16)
    maskb = jnp.where(ab > jnp.bfloat16(0.0),
                      jnp.bfloat16(1.0), jnp.bfloat16(0.0))
    mask8_ref[...] = maskb.astype(jnp.int8)
    parts = []
    for h in range(_H):
        wh_aug = whb_ref[h]
        wh_blk = wh_ref[h, pl.ds(i * _BR, _BR), :]
        a1 = as_ref[h, :_NHID, :]
        a2 = as_ref[h, _NHID:, :]
        f1 = jnp.dot(wh_blk, a1, preferred_element_type=jnp.float32)
        f2 = jax.lax.dot_general(a2, wh_ref[h],
                                 (((0,), (1,)), ((), ())),
                                 preferred_element_type=jnp.float32)
        hp = _masked_softmax_matmul(maskb, f1, f2, wh_aug)
        parts.append(_elu(hp))
    hcat = jnp.concatenate(parts, axis=1)
    who = jnp.dot(hcat, wout_ref[...], preferred_element_type=jnp.float32)
    who_ref[...] = who
    whob_ref[...] = _aug(who).astype(jnp.bfloat16)


def _layer2_body(mask8_ref, who_ref, whob_ref, aout_ref, h2_ref):
    i = pl.program_id(0)
    maskb = mask8_ref[...].astype(jnp.bfloat16)
    who_aug = whob_ref[...]
    who_blk = who_ref[pl.ds(i * _BR, _BR), :]
    a1 = aout_ref[:_NHID, :]
    a2 = aout_ref[_NHID:, :]
    f1 = jnp.dot(who_blk, a1, preferred_element_type=jnp.float32)
    f2 = jax.lax.dot_general(a2, who_ref[...],
                             (((0,), (1,)), ((), ())),
                             preferred_element_type=jnp.float32)
    hp = _masked_softmax_matmul(maskb, f1, f2, who_aug)
    h2_ref[...] = _elu(hp)


def _score_body(p1_ref, p2_ref, h2_ref, w_ref, out_ref):
    h2 = h2_ref[...]
    e1 = jnp.dot(p1_ref[...], h2, preferred_element_type=jnp.float32)
    e2 = jnp.dot(p2_ref[...], h2, preferred_element_type=jnp.float32)
    t = jnp.dot(e1, w_ref[...], preferred_element_type=jnp.float32)
    out_ref[...] = jnp.sum(t * e2, axis=1, keepdims=True)


def kernel(x, adj, pair1_map, pair2_map, Ws, As, W_out, A_out, weight):
    wh, whb = pl.pallas_call(
        _proj_body,
        grid=(_N // _BR,),
        in_specs=[
            pl.BlockSpec((_BR, _NFEAT), lambda i: (i, 0)),
            pl.BlockSpec((_H, _NFEAT, _NHID), lambda i: (0, 0, 0)),
        ],
        out_specs=[
            pl.BlockSpec((_H, _BR, _NHID), lambda i: (0, i, 0)),
            pl.BlockSpec((_H, _BR, 2 * _NHID), lambda i: (0, i, 0)),
        ],
        out_shape=[
            jax.ShapeDtypeStruct((_H, _N, _NHID), jnp.float32),
            jax.ShapeDtypeStruct((_H, _N, 2 * _NHID), jnp.bfloat16),
        ],
    )(x, Ws)

    who, whob, mask8 = pl.pallas_call(
        _layer1_body,
        grid=(_N // _BR,),
        in_specs=[
            pl.BlockSpec((_BR, _N), lambda i: (i, 0)),
            pl.BlockSpec((_H, _N, _NHID), lambda i: (0, 0, 0)),
            pl.BlockSpec((_H, _N, 2 * _NHID), lambda i: (0, 0, 0)),
            pl.BlockSpec((_H, 2 * _NHID, 1), lambda i: (0, 0, 0)),
            pl.BlockSpec((_H * _NHID, _NHID), lambda i: (0, 0)),
        ],
        out_specs=[
            pl.BlockSpec((_BR, _NHID), lambda i: (i, 0)),
            pl.BlockSpec((_BR, 2 * _NHID), lambda i: (i, 0)),
            pl.BlockSpec((_BR, _N), lambda i: (i, 0)),
        ],
        out_shape=[
            jax.ShapeDtypeStruct((_N, _NHID), jnp.float32),
            jax.ShapeDtypeStruct((_N, 2 * _NHID), jnp.bfloat16),
            jax.ShapeDtypeStruct((_N, _N), jnp.int8),
        ],
    )(adj, wh, whb, As, W_out)

    h2 = pl.pallas_call(
        _layer2_body,
        grid=(_N // _BR,),
        in_specs=[
            pl.BlockSpec((_BR, _N), lambda i: (i, 0)),
            pl.BlockSpec((_N, _NHID), lambda i: (0, 0)),
            pl.BlockSpec((_N, 2 * _NHID), lambda i: (0, 0)),
            pl.BlockSpec((2 * _NHID, 1), lambda i: (0, 0)),
        ],
        out_specs=pl.BlockSpec((_BR, _NHID), lambda i: (i, 0)),
        out_shape=jax.ShapeDtypeStruct((_N, _NHID), jnp.float32),
    )(mask8, who, whob, A_out)

    scores = pl.pallas_call(
        _score_body,
        grid=(_P // _BP,),
        in_specs=[
            pl.BlockSpec((_BP, _N), lambda i: (i, 0)),
            pl.BlockSpec((_BP, _N), lambda i: (i, 0)),
            pl.BlockSpec((_N, _NHID), lambda i: (0, 0)),
            pl.BlockSpec((_NHID, _NHID), lambda i: (0, 0)),
        ],
        out_specs=pl.BlockSpec((_BP, 1), lambda i: (i, 0)),
        out_shape=jax.ShapeDtypeStruct((_P, 1), jnp.float32),
    )(pair1_map, pair2_map, h2, weight)

    return scores.reshape(_P)
